# Initial kernel scaffold; baseline (speedup 1.0000x reference)
import jax
import jax.numpy as jnp
from jax import lax
from jax.experimental import pallas as pl
from jax.experimental.pallas import tpu as pltpu
from jax.experimental.pallas import tpu_sc as plsc

N = 10000
E = 320000
H = 8
F = 16
HF = H * F
ROW = HF + 16
TAB = ROW + 16

CB = 128
NCHUNK = E // CB
NTILE = 32
RPT = N // 16
MBLK = 400


def _matmul_body(x_ref, w_ref, o_ref):
    o_ref[...] = jnp.dot(x_ref[...], w_ref[...],
                         preferred_element_type=jnp.float32)


def _prep(x, wcat):
    k = x.shape[1]
    return pl.pallas_call(
        _matmul_body,
        grid=(N // MBLK,),
        in_specs=[pl.BlockSpec((MBLK, k), lambda i: (i, 0)),
                  pl.BlockSpec((k, TAB), lambda i: (0, 0))],
        out_specs=pl.BlockSpec((MBLK, TAB), lambda i: (i, 0)),
        out_shape=jax.ShapeDtypeStruct((N, TAB), jnp.float32),
    )(x, wcat)


def _mid_body(p0_ref, p1_ref, bmat_ref, bflat_ref, mmean_ref, w2_ref, o_ref):
    acc = p0_ref[...] + p1_ref[...]
    den = jnp.dot(acc[:, HF:ROW], bmat_ref[...],
                  preferred_element_type=jnp.float32)
    r = 1.0 / (den + 1e-9)
    x1 = jnp.dot(acc[:, :HF] * r + bflat_ref[...], mmean_ref[...],
                 preferred_element_type=jnp.float32)
    x2 = jnp.maximum(x1, 0.0)
    o_ref[...] = jnp.dot(x2, w2_ref[...], preferred_element_type=jnp.float32)


def _mid(p0, p1, bmat, bflat, mmean, wcat2):
    return pl.pallas_call(
        _mid_body,
        grid=(N // MBLK,),
        in_specs=[pl.BlockSpec((MBLK, ROW), lambda i: (i, 0)),
                  pl.BlockSpec((MBLK, ROW), lambda i: (i, 0)),
                  pl.BlockSpec((16, HF), lambda i: (0, 0)),
                  pl.BlockSpec((1, HF), lambda i: (0, 0)),
                  pl.BlockSpec((HF, F), lambda i: (0, 0)),
                  pl.BlockSpec((F, TAB), lambda i: (0, 0))],
        out_specs=pl.BlockSpec((MBLK, TAB), lambda i: (i, 0)),
        out_shape=jax.ShapeDtypeStruct((N, TAB), jnp.float32),
    )(p0, p1, bmat, bflat, mmean, wcat2)


def _final_body(p0_ref, p1_ref, bmat_ref, bflat_ref, mmean_ref, o_ref):
    acc = p0_ref[...] + p1_ref[...]
    den = jnp.dot(acc[:, HF:ROW], bmat_ref[...],
                  preferred_element_type=jnp.float32)
    r = 1.0 / (den + 1e-9)
    o_ref[...] = jnp.dot(acc[:, :HF] * r + bflat_ref[...], mmean_ref[...],
                         preferred_element_type=jnp.float32)


def _final(p0, p1, bmat, bflat, mmean):
    return pl.pallas_call(
        _final_body,
        grid=(N // MBLK,),
        in_specs=[pl.BlockSpec((MBLK, ROW), lambda i: (i, 0)),
                  pl.BlockSpec((MBLK, ROW), lambda i: (i, 0)),
                  pl.BlockSpec((16, HF), lambda i: (0, 0)),
                  pl.BlockSpec((1, HF), lambda i: (0, 0)),
                  pl.BlockSpec((HF, F), lambda i: (0, 0))],
        out_specs=pl.BlockSpec((MBLK, F), lambda i: (i, 0)),
        out_shape=jax.ShapeDtypeStruct((N, F), jnp.float32),
    )(p0, p1, bmat, bflat, mmean)


def _bcast_lane(v, g):
    return lax.gather(
        v, jnp.full((16, 1), g, jnp.int32),
        lax.GatherDimensionNumbers(offset_dims=(), collapsed_slice_dims=(0,),
                                   start_index_map=(0,)),
        (1,), mode=lax.GatherScatterMode.PROMISE_IN_BOUNDS)


def _edge_pass(src, dst, stab, ertab):
    mesh = plsc.VectorSubcoreMesh(core_axis_name="c", subcore_axis_name="s")

    def body(src_hbm, dst_hbm, stab_hbm, ertab_hbm, out_hbm,
             acc, idx_s, idx_d, rows, ers, zbuf):
        c = lax.axis_index("c")
        s = lax.axis_index("s")
        wid = c * 16 + s

        def zrow(j, _):
            for kk in range(ROW // 16):
                zbuf[j, pl.ds(kk * 16, 16)] = jnp.zeros((16,), jnp.float32)
            return 0
        lax.fori_loop(0, 125, zrow, 0)
        for rblk in range(RPT // 125):
            pltpu.sync_copy(zbuf, acc.at[pl.ds(s * RPT + rblk * 125, 125)])

        plsc.subcore_barrier()

        nt = jnp.where(wid < NCHUNK % NTILE,
                       NCHUNK // NTILE + 1, NCHUNK // NTILE)

        def chunk_body(k, _):
            chunk = k * NTILE + wid
            base = pl.multiple_of(chunk * CB, CB)
            pltpu.sync_copy(src_hbm.at[pl.ds(base, CB)], idx_s)
            pltpu.sync_copy(dst_hbm.at[pl.ds(base, CB)], idx_d)
            pltpu.sync_copy(stab_hbm.at[idx_s], rows)
            pltpu.sync_copy(ertab_hbm.at[idx_d], ers)

            def edge(i, _):
                el = rows[i, pl.ds(HF, 16)]
                er = ers[i, pl.ds(0, 16)]
                e = el + er
                e = jnp.where(e > 0.0, e, 0.2 * e)
                ee = jnp.exp(e)
                for g in range(H):
                    hv = rows[i, pl.ds(g * 16, 16)]
                    rows[i, pl.ds(g * 16, 16)] = hv * _bcast_lane(ee, g)
                rows[i, pl.ds(HF, 16)] = ee
                return 0
            lax.fori_loop(0, CB, edge, 0)

            pltpu.sync_copy(rows, acc.at[idx_d], add=True)
            return 0
        lax.fori_loop(0, nt, chunk_body, 0)

        plsc.subcore_barrier()

        pltpu.sync_copy(acc.at[pl.ds(s * RPT, RPT)],
                        out_hbm.at[c, pl.ds(s * RPT, RPT)])

    return pl.kernel(
        body,
        out_type=jax.ShapeDtypeStruct((2, N, ROW), jnp.float32),
        mesh=mesh,
        scratch_types=[
            pltpu.VMEM_SHARED((N, ROW), jnp.float32),
            pltpu.VMEM((CB,), jnp.int32),
            pltpu.VMEM((CB,), jnp.int32),
            pltpu.VMEM((CB, ROW), jnp.float32),
            pltpu.VMEM((CB, 16), jnp.float32),
            pltpu.VMEM((125, ROW), jnp.float32),
        ],
    )(src, dst, stab, ertab)


def _build_wcat(W, attn_l, attn_r):
    k = W.shape[0]
    eye_h = jnp.eye(H, dtype=jnp.float32)
    sel_l = (eye_h[:, None, :] * attn_l.T[None, :, :]).reshape(HF, H)
    sel_r = (eye_h[:, None, :] * attn_r.T[None, :, :]).reshape(HF, H)
    z8 = jnp.zeros((k, 8), jnp.float32)
    return jnp.concatenate([W, W @ sel_l, z8, W @ sel_r, z8], axis=1)


def kernel(emb, edge_index, W1, attn_l1, attn_r1, b1, W2, attn_l2, attn_r2, b2):
    src = edge_index[0].astype(jnp.int32)
    dst = edge_index[1].astype(jnp.int32)

    wcat1 = _build_wcat(W1, attn_l1, attn_r1)
    wcat2 = _build_wcat(W2, attn_l2, attn_r2)
    bmat = jnp.repeat(jnp.eye(16, dtype=jnp.float32)[:, :H], F, axis=1)
    mmean = jnp.tile(jnp.eye(F, dtype=jnp.float32), (H, 1)) / H
    b1f = b1.reshape(1, HF)
    b2f = b2.reshape(1, HF)

    tab1 = _prep(emb, wcat1)
    part1 = _edge_pass(src, dst, tab1[:, :ROW], tab1[:, ROW:])
    tab2 = _mid(part1[0], part1[1], bmat, b1f, mmean, wcat2)
    part2 = _edge_pass(src, dst, tab2[:, :ROW], tab2[:, ROW:])
    return _final(part2[0], part2[1], bmat, b2f, mmean)


# speedup vs baseline: 66.6621x; 66.6621x over previous
import jax
import jax.numpy as jnp
from jax import lax
from jax.experimental import pallas as pl
from jax.experimental.pallas import tpu as pltpu
from jax.experimental.pallas import tpu_sc as plsc

N = 10000
E = 320000
H = 8
F = 16
HF = H * F
ROW = HF + 16
TAB = ROW + 16

CB = 128
NCHUNK = E // CB
NTILE = 32
NP = 10240
RPT = NP // 16
MBLK = 400


def _matmul_body(x_ref, w_ref, o_ref):
    o_ref[...] = jnp.dot(x_ref[...], w_ref[...],
                         preferred_element_type=jnp.float32)


def _prep(x, wcat):
    k = x.shape[1]
    return pl.pallas_call(
        _matmul_body,
        grid=(N // MBLK,),
        in_specs=[pl.BlockSpec((MBLK, k), lambda i: (i, 0)),
                  pl.BlockSpec((k, TAB), lambda i: (0, 0))],
        out_specs=pl.BlockSpec((MBLK, TAB), lambda i: (i, 0)),
        out_shape=jax.ShapeDtypeStruct((N, TAB), jnp.float32),
    )(x, wcat)


def _mid_body(p0_ref, p1_ref, bmat_ref, bflat_ref, mmean_ref, w2_ref, o_ref):
    acc = p0_ref[...] + p1_ref[...]
    den = jnp.dot(acc[:, HF:ROW], bmat_ref[...],
                  preferred_element_type=jnp.float32)
    r = 1.0 / (den + 1e-9)
    x1 = jnp.dot(acc[:, :HF] * r + bflat_ref[...], mmean_ref[...],
                 preferred_element_type=jnp.float32)
    x2 = jnp.maximum(x1, 0.0)
    o_ref[...] = jnp.dot(x2, w2_ref[...], preferred_element_type=jnp.float32)


def _mid(p0, p1, bmat, bflat, mmean, wcat2):
    return pl.pallas_call(
        _mid_body,
        grid=(N // MBLK,),
        in_specs=[pl.BlockSpec((MBLK, ROW), lambda i: (i, 0)),
                  pl.BlockSpec((MBLK, ROW), lambda i: (i, 0)),
                  pl.BlockSpec((16, HF), lambda i: (0, 0)),
                  pl.BlockSpec((1, HF), lambda i: (0, 0)),
                  pl.BlockSpec((HF, F), lambda i: (0, 0)),
                  pl.BlockSpec((F, TAB), lambda i: (0, 0))],
        out_specs=pl.BlockSpec((MBLK, TAB), lambda i: (i, 0)),
        out_shape=jax.ShapeDtypeStruct((N, TAB), jnp.float32),
    )(p0, p1, bmat, bflat, mmean, wcat2)


def _final_body(p0_ref, p1_ref, bmat_ref, bflat_ref, mmean_ref, o_ref):
    acc = p0_ref[...] + p1_ref[...]
    den = jnp.dot(acc[:, HF:ROW], bmat_ref[...],
                  preferred_element_type=jnp.float32)
    r = 1.0 / (den + 1e-9)
    o_ref[...] = jnp.dot(acc[:, :HF] * r + bflat_ref[...], mmean_ref[...],
                         preferred_element_type=jnp.float32)


def _final(p0, p1, bmat, bflat, mmean):
    return pl.pallas_call(
        _final_body,
        grid=(N // MBLK,),
        in_specs=[pl.BlockSpec((MBLK, ROW), lambda i: (i, 0)),
                  pl.BlockSpec((MBLK, ROW), lambda i: (i, 0)),
                  pl.BlockSpec((16, HF), lambda i: (0, 0)),
                  pl.BlockSpec((1, HF), lambda i: (0, 0)),
                  pl.BlockSpec((HF, F), lambda i: (0, 0))],
        out_specs=pl.BlockSpec((MBLK, F), lambda i: (i, 0)),
        out_shape=jax.ShapeDtypeStruct((N, F), jnp.float32),
    )(p0, p1, bmat, bflat, mmean)


def _bcast_lane(v, g):
    return lax.gather(
        v, jnp.full((16, 1), g, jnp.int32),
        lax.GatherDimensionNumbers(offset_dims=(), collapsed_slice_dims=(0,),
                                   start_index_map=(0,)),
        (1,), mode=lax.GatherScatterMode.PROMISE_IN_BOUNDS)


def _edge_pass(src, dst, stab, ertab):
    mesh = plsc.VectorSubcoreMesh(core_axis_name="c", subcore_axis_name="s")

    def body(src_hbm, dst_hbm, stab_hbm, ertab_hbm, out_hbm,
             acc, idx_s, idx_d, rows, ers):
        c = lax.axis_index("c")
        s = lax.axis_index("s")
        wid = c * 16 + s

        def zrow(j, _):
            for kk in range(ROW // 16):
                rows[j, pl.ds(kk * 16, 16)] = jnp.zeros((16,), jnp.float32)
            return 0
        lax.fori_loop(0, 128, zrow, 0)
        for rblk in range(RPT // 128):
            pltpu.sync_copy(rows, acc.at[pl.ds(s * RPT + rblk * 128, 128)])

        plsc.subcore_barrier()

        nt = jnp.where(wid < NCHUNK % NTILE,
                       NCHUNK // NTILE + 1, NCHUNK // NTILE)

        def chunk_body(k, _):
            chunk = k * NTILE + wid
            base = pl.multiple_of(chunk * CB, CB)
            pltpu.sync_copy(src_hbm.at[pl.ds(base, CB)], idx_s)
            pltpu.sync_copy(dst_hbm.at[pl.ds(base, CB)], idx_d)
            pltpu.sync_copy(stab_hbm.at[idx_s], rows)
            pltpu.sync_copy(ertab_hbm.at[idx_d], ers)

            def edge(i, _):
                el = rows[i, pl.ds(HF, 16)]
                er = ers[i, pl.ds(0, 16)]
                e = el + er
                e = jnp.where(e > 0.0, e, 0.2 * e)
                ee = jnp.exp(e)
                for g in range(H):
                    hv = rows[i, pl.ds(g * 16, 16)]
                    rows[i, pl.ds(g * 16, 16)] = hv * _bcast_lane(ee, g)
                rows[i, pl.ds(HF, 16)] = ee
                return 0
            lax.fori_loop(0, CB, edge, 0)

            pltpu.sync_copy(rows, acc.at[idx_d], add=True)
            return 0
        lax.fori_loop(0, nt, chunk_body, 0)

        plsc.subcore_barrier()

        pltpu.sync_copy(acc.at[pl.ds(s * RPT, RPT)],
                        out_hbm.at[c, pl.ds(s * RPT, RPT)])

    return pl.kernel(
        body,
        out_type=jax.ShapeDtypeStruct((2, NP, ROW), jnp.float32),
        mesh=mesh,
        compiler_params=pltpu.CompilerParams(use_tc_tiling_on_sc=False),
        scratch_types=[
            pltpu.VMEM_SHARED((NP, ROW), jnp.float32),
            pltpu.VMEM((CB,), jnp.int32),
            pltpu.VMEM((CB,), jnp.int32),
            pltpu.VMEM((CB, ROW), jnp.float32),
            pltpu.VMEM((CB, 16), jnp.float32),
        ],
    )(src, dst, stab, ertab)


def _build_wcat(W, attn_l, attn_r):
    k = W.shape[0]
    eye_h = jnp.eye(H, dtype=jnp.float32)
    sel_l = (eye_h[:, None, :] * attn_l.T[None, :, :]).reshape(HF, H)
    sel_r = (eye_h[:, None, :] * attn_r.T[None, :, :]).reshape(HF, H)
    z8 = jnp.zeros((k, 8), jnp.float32)
    return jnp.concatenate([W, W @ sel_l, z8, W @ sel_r, z8], axis=1)


def kernel(emb, edge_index, W1, attn_l1, attn_r1, b1, W2, attn_l2, attn_r2, b2):
    src = edge_index[0].astype(jnp.int32)
    dst = edge_index[1].astype(jnp.int32)

    wcat1 = _build_wcat(W1, attn_l1, attn_r1)
    wcat2 = _build_wcat(W2, attn_l2, attn_r2)
    bmat = jnp.repeat(jnp.eye(16, dtype=jnp.float32)[:, :H], F, axis=1)
    mmean = jnp.tile(jnp.eye(F, dtype=jnp.float32), (H, 1)) / H
    b1f = b1.reshape(1, HF)
    b2f = b2.reshape(1, HF)

    tab1 = _prep(emb, wcat1)
    part1 = _edge_pass(src, dst, tab1[:, :ROW], tab1[:, ROW:])
    tab2 = _mid(part1[0, :N], part1[1, :N], bmat, b1f, mmean, wcat2)
    part2 = _edge_pass(src, dst, tab2[:, :ROW], tab2[:, ROW:])
    return _final(part2[0, :N], part2[1, :N], bmat, b2f, mmean)


# speedup vs baseline: 117.7743x; 1.7667x over previous
import jax
import jax.numpy as jnp
from jax import lax
from jax.experimental import pallas as pl
from jax.experimental.pallas import tpu as pltpu
from jax.experimental.pallas import tpu_sc as plsc

N = 10000
E = 320000
H = 8
F = 16
HF = H * F
ROW = HF + 16
TAB = ROW + 16

CB = 80
NTILE = 32
EPT = E // NTILE
NCHT = EPT // CB
NP = 10240
RPT = NP // 16
MBLK = 400


def _matmul_body(x_ref, wa_ref, wb_ref, o1_ref, o2_ref):
    x = x_ref[...]
    o1_ref[...] = jnp.dot(x, wa_ref[...], preferred_element_type=jnp.float32)
    o2_ref[...] = jnp.dot(x, wb_ref[...], preferred_element_type=jnp.float32)


def _prep(x, wcat):
    k = x.shape[1]
    return pl.pallas_call(
        _matmul_body,
        grid=(N // MBLK,),
        in_specs=[pl.BlockSpec((MBLK, k), lambda i: (i, 0)),
                  pl.BlockSpec((k, ROW), lambda i: (0, 0)),
                  pl.BlockSpec((k, 16), lambda i: (0, 0))],
        out_specs=[pl.BlockSpec((MBLK, ROW), lambda i: (i, 0)),
                   pl.BlockSpec((MBLK, 16), lambda i: (i, 0))],
        out_shape=[jax.ShapeDtypeStruct((N, ROW), jnp.float32),
                   jax.ShapeDtypeStruct((N, 16), jnp.float32)],
    )(x, wcat[:, :ROW], wcat[:, ROW:])


def _mid_body(p0_ref, p1_ref, bmat_ref, bflat_ref, mmean_ref, wa_ref, wb_ref,
              o1_ref, o2_ref):
    acc = p0_ref[...] + p1_ref[...]
    den = jnp.dot(acc[:, HF:ROW], bmat_ref[...],
                  preferred_element_type=jnp.float32)
    r = 1.0 / (den + 1e-9)
    x1 = jnp.dot(acc[:, :HF] * r + bflat_ref[...], mmean_ref[...],
                 preferred_element_type=jnp.float32)
    x2 = jnp.maximum(x1, 0.0)
    o1_ref[...] = jnp.dot(x2, wa_ref[...], preferred_element_type=jnp.float32)
    o2_ref[...] = jnp.dot(x2, wb_ref[...], preferred_element_type=jnp.float32)


def _mid(p0, p1, bmat, bflat, mmean, wcat2):
    return pl.pallas_call(
        _mid_body,
        grid=(N // MBLK,),
        in_specs=[pl.BlockSpec((MBLK, ROW), lambda i: (i, 0)),
                  pl.BlockSpec((MBLK, ROW), lambda i: (i, 0)),
                  pl.BlockSpec((16, HF), lambda i: (0, 0)),
                  pl.BlockSpec((1, HF), lambda i: (0, 0)),
                  pl.BlockSpec((HF, F), lambda i: (0, 0)),
                  pl.BlockSpec((F, ROW), lambda i: (0, 0)),
                  pl.BlockSpec((F, 16), lambda i: (0, 0))],
        out_specs=[pl.BlockSpec((MBLK, ROW), lambda i: (i, 0)),
                   pl.BlockSpec((MBLK, 16), lambda i: (i, 0))],
        out_shape=[jax.ShapeDtypeStruct((N, ROW), jnp.float32),
                   jax.ShapeDtypeStruct((N, 16), jnp.float32)],
    )(p0, p1, bmat, bflat, mmean, wcat2[:, :ROW], wcat2[:, ROW:])


def _final_body(p0_ref, p1_ref, bmat_ref, bflat_ref, mmean_ref, o_ref):
    acc = p0_ref[...] + p1_ref[...]
    den = jnp.dot(acc[:, HF:ROW], bmat_ref[...],
                  preferred_element_type=jnp.float32)
    r = 1.0 / (den + 1e-9)
    o_ref[...] = jnp.dot(acc[:, :HF] * r + bflat_ref[...], mmean_ref[...],
                         preferred_element_type=jnp.float32)


def _final(p0, p1, bmat, bflat, mmean):
    return pl.pallas_call(
        _final_body,
        grid=(N // MBLK,),
        in_specs=[pl.BlockSpec((MBLK, ROW), lambda i: (i, 0)),
                  pl.BlockSpec((MBLK, ROW), lambda i: (i, 0)),
                  pl.BlockSpec((16, HF), lambda i: (0, 0)),
                  pl.BlockSpec((1, HF), lambda i: (0, 0)),
                  pl.BlockSpec((HF, F), lambda i: (0, 0))],
        out_specs=pl.BlockSpec((MBLK, F), lambda i: (i, 0)),
        out_shape=jax.ShapeDtypeStruct((N, F), jnp.float32),
    )(p0, p1, bmat, bflat, mmean)


def _bcast_lane(v, g):
    return lax.gather(
        v, jnp.full((16, 1), g, jnp.int32),
        lax.GatherDimensionNumbers(offset_dims=(), collapsed_slice_dims=(0,),
                                   start_index_map=(0,)),
        (1,), mode=lax.GatherScatterMode.PROMISE_IN_BOUNDS)


def _edge_pass(src, dst, stab, ertab):
    mesh = plsc.VectorSubcoreMesh(core_axis_name="c", subcore_axis_name="s")

    def body(src_hbm, dst_hbm, stab_hbm, ertab_hbm, out_hbm,
             acc, idx_s0, idx_d0, idx_s1, idx_d1, rows0, rows1, ers0, ers1,
             sem_r0, sem_r1, sem_e0, sem_e1):
        idx_s = (idx_s0, idx_s1)
        idx_d = (idx_d0, idx_d1)
        rows = (rows0, rows1)
        ers = (ers0, ers1)
        sem_r = (sem_r0, sem_r1)
        sem_e = (sem_e0, sem_e1)
        c = lax.axis_index("c")
        s = lax.axis_index("s")
        wid = c * 16 + s
        tbase = wid * EPT

        def zrow(j, _):
            for kk in range(ROW // 16):
                rows0[j, pl.ds(kk * 16, 16)] = jnp.zeros((16,), jnp.float32)
            return 0
        lax.fori_loop(0, CB, zrow, 0)
        for rblk in range(RPT // CB):
            pltpu.sync_copy(rows0, acc.at[pl.ds(s * RPT + rblk * CB, CB)])

        plsc.subcore_barrier()

        def start(k, b):
            base = pl.multiple_of(tbase + k * CB, 8)
            pltpu.sync_copy(src_hbm.at[pl.ds(base, CB)], idx_s[b])
            pltpu.sync_copy(dst_hbm.at[pl.ds(base, CB)], idx_d[b])
            pltpu.async_copy(stab_hbm.at[idx_s[b]], rows[b], sem_r[b])
            pltpu.async_copy(ertab_hbm.at[idx_d[b]], ers[b], sem_e[b])

        def consume(b):
            rb, eb = rows[b], ers[b]
            pltpu.make_async_copy(stab_hbm.at[idx_s[b]], rb, sem_r[b]).wait()
            pltpu.make_async_copy(ertab_hbm.at[idx_d[b]], eb, sem_e[b]).wait()

            @plsc.parallel_loop(0, CB, unroll=4)
            def _edge(i):
                el = rb[i, pl.ds(HF, 16)]
                er = eb[i, pl.ds(0, 16)]
                e = el + er
                e = jnp.where(e > 0.0, e, 0.2 * e)
                ee = jnp.exp(e)
                for g in range(H):
                    hv = rb[i, pl.ds(g * 16, 16)]
                    rb[i, pl.ds(g * 16, 16)] = hv * _bcast_lane(ee, g)
                rb[i, pl.ds(HF, 16)] = ee

            pltpu.sync_copy(rb, acc.at[idx_d[b]], add=True)

        start(0, 0)
        start(1, 1)

        def pair(i, _):
            k0 = 2 * i
            consume(0)

            @pl.when(k0 + 2 < NCHT)
            def _():
                start(k0 + 2, 0)

            consume(1)

            @pl.when(k0 + 3 < NCHT)
            def _():
                start(k0 + 3, 1)
            return 0
        lax.fori_loop(0, NCHT // 2, pair, 0)
        consume(0)

        plsc.subcore_barrier()

        pltpu.sync_copy(acc.at[pl.ds(s * RPT, RPT)],
                        out_hbm.at[c, pl.ds(s * RPT, RPT)])

    return pl.kernel(
        body,
        out_type=jax.ShapeDtypeStruct((2, NP, ROW), jnp.float32),
        mesh=mesh,
        compiler_params=pltpu.CompilerParams(use_tc_tiling_on_sc=False),
        scratch_types=[
            pltpu.VMEM_SHARED((NP, ROW), jnp.float32),
            pltpu.VMEM((CB,), jnp.int32),
            pltpu.VMEM((CB,), jnp.int32),
            pltpu.VMEM((CB,), jnp.int32),
            pltpu.VMEM((CB,), jnp.int32),
            pltpu.VMEM((CB, ROW), jnp.float32),
            pltpu.VMEM((CB, ROW), jnp.float32),
            pltpu.VMEM((CB, 16), jnp.float32),
            pltpu.VMEM((CB, 16), jnp.float32),
            pltpu.SemaphoreType.DMA,
            pltpu.SemaphoreType.DMA,
            pltpu.SemaphoreType.DMA,
            pltpu.SemaphoreType.DMA,
        ],
    )(src, dst, stab, ertab)


def _build_wcat(W, attn_l, attn_r):
    k = W.shape[0]
    eye_h = jnp.eye(H, dtype=jnp.float32)
    sel_l = (eye_h[:, None, :] * attn_l.T[None, :, :]).reshape(HF, H)
    sel_r = (eye_h[:, None, :] * attn_r.T[None, :, :]).reshape(HF, H)
    z8 = jnp.zeros((k, 8), jnp.float32)
    return jnp.concatenate([W, W @ sel_l, z8, W @ sel_r, z8], axis=1)


def kernel(emb, edge_index, W1, attn_l1, attn_r1, b1, W2, attn_l2, attn_r2, b2):
    src = edge_index[0].astype(jnp.int32)
    dst = edge_index[1].astype(jnp.int32)

    wcat1 = _build_wcat(W1, attn_l1, attn_r1)
    wcat2 = _build_wcat(W2, attn_l2, attn_r2)
    bmat = jnp.repeat(jnp.eye(16, dtype=jnp.float32)[:, :H], F, axis=1)
    mmean = jnp.tile(jnp.eye(F, dtype=jnp.float32), (H, 1)) / H
    b1f = b1.reshape(1, HF)
    b2f = b2.reshape(1, HF)

    stab1, ertab1 = _prep(emb, wcat1)
    part1 = _edge_pass(src, dst, stab1, ertab1)
    stab2, ertab2 = _mid(part1[0, :N], part1[1, :N], bmat, b1f, mmean, wcat2)
    part2 = _edge_pass(src, dst, stab2, ertab2)
    return _final(part2[0, :N], part2[1, :N], bmat, b2f, mmean)


# speedup vs baseline: 157.5640x; 1.3378x over previous
import jax
import jax.numpy as jnp
from jax import lax
from jax.experimental import pallas as pl
from jax.experimental.pallas import tpu as pltpu
from jax.experimental.pallas import tpu_sc as plsc

N = 10000
E = 320000
H = 8
F = 16
HF = H * F
ROW = HF + 16
TAB = ROW + 16

CB = 80
NTILE = 32
EPT = E // NTILE
NCHT = EPT // CB
NP = 10240
RPT = NP // 16
MBLK = 400


def _matmul_body(x_ref, wa_ref, wb_ref, o1_ref, o2_ref):
    x = x_ref[...]
    o1_ref[...] = jnp.dot(x, wa_ref[...], preferred_element_type=jnp.float32)
    o2_ref[...] = jnp.dot(x, wb_ref[...], preferred_element_type=jnp.float32)


def _prep(x, wcat):
    k = x.shape[1]
    return pl.pallas_call(
        _matmul_body,
        grid=(N // MBLK,),
        in_specs=[pl.BlockSpec((MBLK, k), lambda i: (i, 0)),
                  pl.BlockSpec((k, ROW), lambda i: (0, 0)),
                  pl.BlockSpec((k, 16), lambda i: (0, 0))],
        out_specs=[pl.BlockSpec((MBLK, ROW), lambda i: (i, 0)),
                   pl.BlockSpec((MBLK, 16), lambda i: (i, 0))],
        out_shape=[jax.ShapeDtypeStruct((N, ROW), jnp.float32),
                   jax.ShapeDtypeStruct((N, 16), jnp.float32)],
    )(x, wcat[:, :ROW], wcat[:, ROW:])


def _mid_body(p0_ref, p1_ref, bmat_ref, bflat_ref, mmean_ref, wa_ref, wb_ref,
              o1_ref, o2_ref):
    acc = p0_ref[...] + p1_ref[...]
    den = jnp.dot(acc[:, HF:ROW], bmat_ref[...],
                  preferred_element_type=jnp.float32)
    r = 1.0 / (den + 1e-9)
    x1 = jnp.dot(acc[:, :HF] * r + bflat_ref[...], mmean_ref[...],
                 preferred_element_type=jnp.float32)
    x2 = jnp.maximum(x1, 0.0)
    o1_ref[...] = jnp.dot(x2, wa_ref[...], preferred_element_type=jnp.float32)
    o2_ref[...] = jnp.dot(x2, wb_ref[...], preferred_element_type=jnp.float32)


def _mid(p0, p1, bmat, bflat, mmean, wcat2):
    return pl.pallas_call(
        _mid_body,
        grid=(N // MBLK,),
        in_specs=[pl.BlockSpec((MBLK, ROW), lambda i: (i, 0)),
                  pl.BlockSpec((MBLK, ROW), lambda i: (i, 0)),
                  pl.BlockSpec((16, HF), lambda i: (0, 0)),
                  pl.BlockSpec((1, HF), lambda i: (0, 0)),
                  pl.BlockSpec((HF, F), lambda i: (0, 0)),
                  pl.BlockSpec((F, ROW), lambda i: (0, 0)),
                  pl.BlockSpec((F, 16), lambda i: (0, 0))],
        out_specs=[pl.BlockSpec((MBLK, ROW), lambda i: (i, 0)),
                   pl.BlockSpec((MBLK, 16), lambda i: (i, 0))],
        out_shape=[jax.ShapeDtypeStruct((N, ROW), jnp.float32),
                   jax.ShapeDtypeStruct((N, 16), jnp.float32)],
    )(p0, p1, bmat, bflat, mmean, wcat2[:, :ROW], wcat2[:, ROW:])


def _final_body(p0_ref, p1_ref, bmat_ref, bflat_ref, mmean_ref, o_ref):
    acc = p0_ref[...] + p1_ref[...]
    den = jnp.dot(acc[:, HF:ROW], bmat_ref[...],
                  preferred_element_type=jnp.float32)
    r = 1.0 / (den + 1e-9)
    o_ref[...] = jnp.dot(acc[:, :HF] * r + bflat_ref[...], mmean_ref[...],
                         preferred_element_type=jnp.float32)


def _final(p0, p1, bmat, bflat, mmean):
    return pl.pallas_call(
        _final_body,
        grid=(N // MBLK,),
        in_specs=[pl.BlockSpec((MBLK, ROW), lambda i: (i, 0)),
                  pl.BlockSpec((MBLK, ROW), lambda i: (i, 0)),
                  pl.BlockSpec((16, HF), lambda i: (0, 0)),
                  pl.BlockSpec((1, HF), lambda i: (0, 0)),
                  pl.BlockSpec((HF, F), lambda i: (0, 0))],
        out_specs=pl.BlockSpec((MBLK, F), lambda i: (i, 0)),
        out_shape=jax.ShapeDtypeStruct((N, F), jnp.float32),
    )(p0, p1, bmat, bflat, mmean)


def _bcast_lane(v, g):
    return lax.gather(
        v, jnp.full((16, 1), g, jnp.int32),
        lax.GatherDimensionNumbers(offset_dims=(), collapsed_slice_dims=(0,),
                                   start_index_map=(0,)),
        (1,), mode=lax.GatherScatterMode.PROMISE_IN_BOUNDS)


def _edge_pass(src, dst, stab, ertab):
    mesh = plsc.VectorSubcoreMesh(core_axis_name="c", subcore_axis_name="s")

    def body(src_hbm, dst_hbm, stab_hbm, ertab_hbm, out_hbm,
             acc,
             sidx0, sidx1, sidx2, sidx3, didx0, didx1, didx2, didx3,
             rows0, rows1, ers0, ers1,
             ssi0, ssi1, ssi2, ssi3, sdi0, sdi1, sdi2, sdi3,
             sr0, sr1, se0, se1):
        sidx = (sidx0, sidx1, sidx2, sidx3)
        didx = (didx0, didx1, didx2, didx3)
        ssi = (ssi0, ssi1, ssi2, ssi3)
        sdi = (sdi0, sdi1, sdi2, sdi3)
        rows = (rows0, rows1)
        ers = (ers0, ers1)
        sr = (sr0, sr1)
        se = (se0, se1)
        c = lax.axis_index("c")
        s = lax.axis_index("s")
        wid = c * 16 + s
        tbase = wid * EPT

        def idx_start(k, b):
            base = pl.multiple_of(tbase + k * CB, 8)
            pltpu.async_copy(src_hbm.at[pl.ds(base, CB)], sidx[b], ssi[b])
            pltpu.async_copy(dst_hbm.at[pl.ds(base, CB)], didx[b], sdi[b])

        def gather_start(b2, bi):
            pltpu.make_async_copy(src_hbm.at[pl.ds(0, CB)], sidx[bi],
                                  ssi[bi]).wait()
            pltpu.make_async_copy(dst_hbm.at[pl.ds(0, CB)], didx[bi],
                                  sdi[bi]).wait()
            pltpu.async_copy(stab_hbm.at[sidx[bi]], rows[b2], sr[b2])
            pltpu.async_copy(ertab_hbm.at[didx[bi]], ers[b2], se[b2])

        def consume(b2, bi):
            rb, eb = rows[b2], ers[b2]
            pltpu.make_async_copy(stab_hbm.at[sidx[bi]], rb, sr[b2]).wait()
            pltpu.make_async_copy(ertab_hbm.at[didx[bi]], eb, se[b2]).wait()

            @plsc.parallel_loop(0, CB, unroll=4)
            def _edge(i):
                el = rb[i, pl.ds(HF, 16)]
                er = eb[i, pl.ds(0, 16)]
                e = el + er
                e = jnp.where(e > 0.0, e, 0.2 * e)
                ee = jnp.exp(e)
                for g in range(H):
                    hv = rb[i, pl.ds(g * 16, 16)]
                    rb[i, pl.ds(g * 16, 16)] = hv * _bcast_lane(ee, g)
                rb[i, pl.ds(HF, 16)] = ee

            pltpu.sync_copy(rb, acc.at[didx[bi]], add=True)

        for k in range(4):
            idx_start(k, k)

        def zrow(j, _):
            for kk in range(ROW // 16):
                rows0[j, pl.ds(kk * 16, 16)] = jnp.zeros((16,), jnp.float32)
            return 0
        lax.fori_loop(0, CB, zrow, 0)
        for rblk in range(RPT // CB):
            pltpu.sync_copy(rows0, acc.at[pl.ds(s * RPT + rblk * CB, CB)])

        gather_start(0, 0)
        gather_start(1, 1)

        plsc.subcore_barrier()

        def slot(k, b):
            consume(b % 2, b)

            @pl.when(k + 4 < NCHT)
            def _():
                idx_start(k + 4, b)

            @pl.when(k + 2 < NCHT)
            def _():
                gather_start(b % 2, (b + 2) % 4)

        def quad(i, _):
            k0 = 4 * i
            for b in range(4):
                slot(k0 + b, b)
            return 0
        lax.fori_loop(0, NCHT // 4, quad, 0)
        slot(NCHT - 1, (NCHT - 1) % 4)

        plsc.subcore_barrier()

        pltpu.sync_copy(acc.at[pl.ds(s * RPT, RPT)],
                        out_hbm.at[c, pl.ds(s * RPT, RPT)])

    return pl.kernel(
        body,
        out_type=jax.ShapeDtypeStruct((2, NP, ROW), jnp.float32),
        mesh=mesh,
        compiler_params=pltpu.CompilerParams(use_tc_tiling_on_sc=False),
        scratch_types=(
            [pltpu.VMEM_SHARED((NP, ROW), jnp.float32)]
            + [pltpu.VMEM((CB,), jnp.int32)] * 8
            + [pltpu.VMEM((CB, ROW), jnp.float32)] * 2
            + [pltpu.VMEM((CB, 16), jnp.float32)] * 2
            + [pltpu.SemaphoreType.DMA] * 12
        ),
    )(src, dst, stab, ertab)


def _build_wcat(W, attn_l, attn_r):
    k = W.shape[0]
    eye_h = jnp.eye(H, dtype=jnp.float32)
    sel_l = (eye_h[:, None, :] * attn_l.T[None, :, :]).reshape(HF, H)
    sel_r = (eye_h[:, None, :] * attn_r.T[None, :, :]).reshape(HF, H)
    z8 = jnp.zeros((k, 8), jnp.float32)
    return jnp.concatenate([W, W @ sel_l, z8, W @ sel_r, z8], axis=1)


def kernel(emb, edge_index, W1, attn_l1, attn_r1, b1, W2, attn_l2, attn_r2, b2):
    src = edge_index[0].astype(jnp.int32)
    dst = edge_index[1].astype(jnp.int32)

    wcat1 = _build_wcat(W1, attn_l1, attn_r1)
    wcat2 = _build_wcat(W2, attn_l2, attn_r2)
    bmat = jnp.repeat(jnp.eye(16, dtype=jnp.float32)[:, :H], F, axis=1)
    mmean = jnp.tile(jnp.eye(F, dtype=jnp.float32), (H, 1)) / H
    b1f = b1.reshape(1, HF)
    b2f = b2.reshape(1, HF)

    stab1, ertab1 = _prep(emb, wcat1)
    part1 = _edge_pass(src, dst, stab1, ertab1)
    stab2, ertab2 = _mid(part1[0, :N], part1[1, :N], bmat, b1f, mmean, wcat2)
    part2 = _edge_pass(src, dst, stab2, ertab2)
    return _final(part2[0, :N], part2[1, :N], bmat, b2f, mmean)


# speedup vs baseline: 162.2418x; 1.0297x over previous
import jax
import jax.numpy as jnp
from jax import lax
from jax.experimental import pallas as pl
from jax.experimental.pallas import tpu as pltpu
from jax.experimental.pallas import tpu_sc as plsc

N = 10000
E = 320000
H = 8
F = 16
HF = H * F
ROW = HF + 16
TAB = ROW + 16

CB = 80
NTILE = 32
EPT = E // NTILE
NCHT = EPT // CB
NP = 10112
RPT = NP // 16
MBLK = 400


def _matmul_body(x_ref, wh_ref, wl_ref, wr_ref, oh_ref, ol_ref, or_ref):
    x = x_ref[...]
    h = jnp.dot(x, wh_ref[...], preferred_element_type=jnp.float32)
    oh_ref[...] = h.astype(jnp.bfloat16)
    ol_ref[...] = jnp.dot(x, wl_ref[...], preferred_element_type=jnp.float32)
    or_ref[...] = jnp.dot(x, wr_ref[...], preferred_element_type=jnp.float32)


def _prep(x, wh, wl, wr):
    k = x.shape[1]
    return pl.pallas_call(
        _matmul_body,
        grid=(N // MBLK,),
        in_specs=[pl.BlockSpec((MBLK, k), lambda i: (i, 0)),
                  pl.BlockSpec((k, HF), lambda i: (0, 0)),
                  pl.BlockSpec((k, 16), lambda i: (0, 0)),
                  pl.BlockSpec((k, 16), lambda i: (0, 0))],
        out_specs=[pl.BlockSpec((MBLK, HF), lambda i: (i, 0)),
                   pl.BlockSpec((MBLK, 16), lambda i: (i, 0)),
                   pl.BlockSpec((MBLK, 16), lambda i: (i, 0))],
        out_shape=[jax.ShapeDtypeStruct((N, HF), jnp.bfloat16),
                   jax.ShapeDtypeStruct((N, 16), jnp.float32),
                   jax.ShapeDtypeStruct((N, 16), jnp.float32)],
    )(x, wh, wl, wr)


def _mid_body(p0_ref, p1_ref, bmat_ref, bflat_ref, mmean_ref,
              wh_ref, wl_ref, wr_ref, oh_ref, ol_ref, or_ref):
    acc = p0_ref[...] + p1_ref[...]
    den = jnp.dot(acc[:, HF:ROW], bmat_ref[...],
                  preferred_element_type=jnp.float32)
    r = 1.0 / (den + 1e-9)
    x1 = jnp.dot(acc[:, :HF] * r + bflat_ref[...], mmean_ref[...],
                 preferred_element_type=jnp.float32)
    x2 = jnp.maximum(x1, 0.0)
    h = jnp.dot(x2, wh_ref[...], preferred_element_type=jnp.float32)
    oh_ref[...] = h.astype(jnp.bfloat16)
    ol_ref[...] = jnp.dot(x2, wl_ref[...], preferred_element_type=jnp.float32)
    or_ref[...] = jnp.dot(x2, wr_ref[...], preferred_element_type=jnp.float32)


def _mid(p0, p1, bmat, bflat, mmean, wh, wl, wr):
    return pl.pallas_call(
        _mid_body,
        grid=(N // MBLK,),
        in_specs=[pl.BlockSpec((MBLK, ROW), lambda i: (i, 0)),
                  pl.BlockSpec((MBLK, ROW), lambda i: (i, 0)),
                  pl.BlockSpec((16, HF), lambda i: (0, 0)),
                  pl.BlockSpec((1, HF), lambda i: (0, 0)),
                  pl.BlockSpec((HF, F), lambda i: (0, 0)),
                  pl.BlockSpec((F, HF), lambda i: (0, 0)),
                  pl.BlockSpec((F, 16), lambda i: (0, 0)),
                  pl.BlockSpec((F, 16), lambda i: (0, 0))],
        out_specs=[pl.BlockSpec((MBLK, HF), lambda i: (i, 0)),
                   pl.BlockSpec((MBLK, 16), lambda i: (i, 0)),
                   pl.BlockSpec((MBLK, 16), lambda i: (i, 0))],
        out_shape=[jax.ShapeDtypeStruct((N, HF), jnp.bfloat16),
                   jax.ShapeDtypeStruct((N, 16), jnp.float32),
                   jax.ShapeDtypeStruct((N, 16), jnp.float32)],
    )(p0, p1, bmat, bflat, mmean, wh, wl, wr)


def _final_body(p0_ref, p1_ref, bmat_ref, bflat_ref, mmean_ref, o_ref):
    acc = p0_ref[...] + p1_ref[...]
    den = jnp.dot(acc[:, HF:ROW], bmat_ref[...],
                  preferred_element_type=jnp.float32)
    r = 1.0 / (den + 1e-9)
    o_ref[...] = jnp.dot(acc[:, :HF] * r + bflat_ref[...], mmean_ref[...],
                         preferred_element_type=jnp.float32)


def _final(p0, p1, bmat, bflat, mmean):
    return pl.pallas_call(
        _final_body,
        grid=(N // MBLK,),
        in_specs=[pl.BlockSpec((MBLK, ROW), lambda i: (i, 0)),
                  pl.BlockSpec((MBLK, ROW), lambda i: (i, 0)),
                  pl.BlockSpec((16, HF), lambda i: (0, 0)),
                  pl.BlockSpec((1, HF), lambda i: (0, 0)),
                  pl.BlockSpec((HF, F), lambda i: (0, 0))],
        out_specs=pl.BlockSpec((MBLK, F), lambda i: (i, 0)),
        out_shape=jax.ShapeDtypeStruct((N, F), jnp.float32),
    )(p0, p1, bmat, bflat, mmean)


def _bcast_lane(v, g):
    return lax.gather(
        v, jnp.full((16, 1), g, jnp.int32),
        lax.GatherDimensionNumbers(offset_dims=(), collapsed_slice_dims=(0,),
                                   start_index_map=(0,)),
        (1,), mode=lax.GatherScatterMode.PROMISE_IN_BOUNDS)


def _edge_pass(src, dst, htab, eltab, ertab):
    mesh = plsc.VectorSubcoreMesh(core_axis_name="c", subcore_axis_name="s")

    def body(src_hbm, dst_hbm, htab_hbm, eltab_hbm, ertab_hbm, out_hbm,
             acc,
             sidx0, sidx1, sidx2, sidx3, didx0, didx1, didx2, didx3,
             hv0, hv1, elv0, elv1, erv0, erv1, msg0, msg1,
             ssi0, ssi1, ssi2, ssi3, sdi0, sdi1, sdi2, sdi3,
             sh0, sh1, sl0, sl1, sre0, sre1):
        sidx = (sidx0, sidx1, sidx2, sidx3)
        didx = (didx0, didx1, didx2, didx3)
        ssi = (ssi0, ssi1, ssi2, ssi3)
        sdi = (sdi0, sdi1, sdi2, sdi3)
        hv = (hv0, hv1)
        elv = (elv0, elv1)
        erv = (erv0, erv1)
        msg = (msg0, msg1)
        sh = (sh0, sh1)
        sl = (sl0, sl1)
        sre = (sre0, sre1)
        c = lax.axis_index("c")
        s = lax.axis_index("s")
        wid = c * 16 + s
        tbase = wid * EPT

        def idx_start(k, b):
            base = pl.multiple_of(tbase + k * CB, 8)
            pltpu.async_copy(src_hbm.at[pl.ds(base, CB)], sidx[b], ssi[b])
            pltpu.async_copy(dst_hbm.at[pl.ds(base, CB)], didx[b], sdi[b])

        def gather_start(b2, bi):
            pltpu.make_async_copy(src_hbm.at[pl.ds(0, CB)], sidx[bi],
                                  ssi[bi]).wait()
            pltpu.make_async_copy(dst_hbm.at[pl.ds(0, CB)], didx[bi],
                                  sdi[bi]).wait()
            pltpu.async_copy(htab_hbm.at[sidx[bi]], hv[b2], sh[b2])
            pltpu.async_copy(eltab_hbm.at[sidx[bi]], elv[b2], sl[b2])
            pltpu.async_copy(ertab_hbm.at[didx[bi]], erv[b2], sre[b2])

        def consume(b2, bi):
            hb, eb, rb, mb = hv[b2], elv[b2], erv[b2], msg[b2]
            pltpu.make_async_copy(htab_hbm.at[sidx[bi]], hb, sh[b2]).wait()
            pltpu.make_async_copy(eltab_hbm.at[sidx[bi]], eb, sl[b2]).wait()
            pltpu.make_async_copy(ertab_hbm.at[didx[bi]], rb, sre[b2]).wait()

            @plsc.parallel_loop(0, CB, unroll=4)
            def _edge(i):
                el = eb[i, pl.ds(0, 16)]
                er = rb[i, pl.ds(0, 16)]
                e = el + er
                e = jnp.where(e > 0.0, e, 0.2 * e)
                ee = jnp.exp(e)
                mb[i, pl.ds(HF, 16)] = ee
                for b4 in range(4):
                    hp = hb[i, pl.ds(32 * b4, 32)]
                    a0, a1 = plsc.unpack(
                        hp, format=plsc.PackFormat.INTERLEAVED)
                    mb[i, pl.ds(32 * b4, 16)] = a0 * _bcast_lane(ee, 2 * b4)
                    mb[i, pl.ds(32 * b4 + 16, 16)] = (
                        a1 * _bcast_lane(ee, 2 * b4 + 1))

            pltpu.sync_copy(mb, acc.at[didx[bi]], add=True)

        for k in range(4):
            idx_start(k, k)

        def zrow(j, _):
            for kk in range(ROW // 16):
                msg0[j, pl.ds(kk * 16, 16)] = jnp.zeros((16,), jnp.float32)
            return 0
        lax.fori_loop(0, CB, zrow, 0)
        for rblk in range(RPT // CB):
            pltpu.sync_copy(msg0, acc.at[pl.ds(s * RPT + rblk * CB, CB)])
        pltpu.sync_copy(msg0.at[pl.ds(0, RPT % CB)],
                        acc.at[pl.ds(s * RPT + (RPT // CB) * CB, RPT % CB)])

        gather_start(0, 0)
        gather_start(1, 1)

        plsc.subcore_barrier()

        def slot(k, b):
            consume(b % 2, b)

            @pl.when(k + 4 < NCHT)
            def _():
                idx_start(k + 4, b)

            @pl.when(k + 2 < NCHT)
            def _():
                gather_start(b % 2, (b + 2) % 4)

        def quad(i, _):
            k0 = 4 * i
            for b in range(4):
                slot(k0 + b, b)
            return 0
        lax.fori_loop(0, NCHT // 4, quad, 0)
        slot(NCHT - 1, (NCHT - 1) % 4)

        plsc.subcore_barrier()

        pltpu.sync_copy(acc.at[pl.ds(s * RPT, RPT)],
                        out_hbm.at[c, pl.ds(s * RPT, RPT)])

    return pl.kernel(
        body,
        out_type=jax.ShapeDtypeStruct((2, NP, ROW), jnp.float32),
        mesh=mesh,
        compiler_params=pltpu.CompilerParams(use_tc_tiling_on_sc=False,
                                             needs_layout_passes=False),
        scratch_types=(
            [pltpu.VMEM_SHARED((NP, ROW), jnp.float32)]
            + [pltpu.VMEM((CB,), jnp.int32)] * 8
            + [pltpu.VMEM((CB, HF), jnp.bfloat16)] * 2
            + [pltpu.VMEM((CB, 16), jnp.float32)] * 4
            + [pltpu.VMEM((CB, ROW), jnp.float32)] * 2
            + [pltpu.SemaphoreType.DMA] * 14
        ),
    )(src, dst, htab, eltab, ertab)


_PERM = [32 * b + j % 2 * 16 + j // 2 for b in range(4) for j in range(32)]


def _build_weights(W, attn_l, attn_r):
    k = W.shape[0]
    eye_h = jnp.eye(H, dtype=jnp.float32)
    sel_l = (eye_h[:, None, :] * attn_l.T[None, :, :]).reshape(HF, H)
    sel_r = (eye_h[:, None, :] * attn_r.T[None, :, :]).reshape(HF, H)
    z8 = jnp.zeros((k, 8), jnp.float32)
    w_el = jnp.concatenate([W @ sel_l, z8], axis=1)
    w_er = jnp.concatenate([W @ sel_r, z8], axis=1)
    return W[:, jnp.array(_PERM)], w_el, w_er


def kernel(emb, edge_index, W1, attn_l1, attn_r1, b1, W2, attn_l2, attn_r2, b2):
    src = edge_index[0].astype(jnp.int32)
    dst = edge_index[1].astype(jnp.int32)

    wh1, wl1, wr1 = _build_weights(W1, attn_l1, attn_r1)
    wh2, wl2, wr2 = _build_weights(W2, attn_l2, attn_r2)
    bmat = jnp.repeat(jnp.eye(16, dtype=jnp.float32)[:, :H], F, axis=1)
    mmean = jnp.tile(jnp.eye(F, dtype=jnp.float32), (H, 1)) / H
    b1f = b1.reshape(1, HF)
    b2f = b2.reshape(1, HF)

    h1, el1, er1 = _prep(emb, wh1, wl1, wr1)
    part1 = _edge_pass(src, dst, h1, el1, er1)
    h2, el2, er2 = _mid(part1[0, :N], part1[1, :N], bmat, b1f, mmean,
                        wh2, wl2, wr2)
    part2 = _edge_pass(src, dst, h2, el2, er2)
    return _final(part2[0, :N], part2[1, :N], bmat, b2f, mmean)


# speedup vs baseline: 179.3194x; 1.1053x over previous
import jax
import jax.numpy as jnp
from jax import lax
from jax.experimental import pallas as pl
from jax.experimental.pallas import tpu as pltpu
from jax.experimental.pallas import tpu_sc as plsc

N = 10000
E = 320000
H = 8
F = 16
HF = H * F
ROW = HF + 16
TAB = ROW + 16

CB = 80
NTILE = 32
EPT = E // NTILE
NCHT = EPT // CB
NP = 10112
RPT = NP // 16
MBLK = 400


def _matmul_body(x_ref, wh_ref, wl_ref, wr_ref, oh_ref, ol_ref, or_ref):
    x = x_ref[...]
    h = jnp.dot(x, wh_ref[...], preferred_element_type=jnp.float32)
    oh_ref[...] = h.astype(jnp.bfloat16)
    ol_ref[...] = jnp.dot(x, wl_ref[...], preferred_element_type=jnp.float32)
    or_ref[...] = jnp.dot(x, wr_ref[...], preferred_element_type=jnp.float32)


def _prep(x, wh, wl, wr):
    k = x.shape[1]
    return pl.pallas_call(
        _matmul_body,
        grid=(N // MBLK,),
        in_specs=[pl.BlockSpec((MBLK, k), lambda i: (i, 0)),
                  pl.BlockSpec((k, HF), lambda i: (0, 0)),
                  pl.BlockSpec((k, 16), lambda i: (0, 0)),
                  pl.BlockSpec((k, 16), lambda i: (0, 0))],
        out_specs=[pl.BlockSpec((MBLK, HF), lambda i: (i, 0)),
                   pl.BlockSpec((MBLK, 16), lambda i: (i, 0)),
                   pl.BlockSpec((MBLK, 16), lambda i: (i, 0))],
        out_shape=[jax.ShapeDtypeStruct((N, HF), jnp.bfloat16),
                   jax.ShapeDtypeStruct((N, 16), jnp.float32),
                   jax.ShapeDtypeStruct((N, 16), jnp.float32)],
    )(x, wh, wl, wr)


def _mid_body(p0_ref, p1_ref, bmat_ref, bflat_ref, mmean_ref,
              wh_ref, wl_ref, wr_ref, oh_ref, ol_ref, or_ref):
    acc = p0_ref[...] + p1_ref[...]
    den = jnp.dot(acc[:, HF:ROW], bmat_ref[...],
                  preferred_element_type=jnp.float32)
    r = 1.0 / (den + 1e-9)
    x1 = jnp.dot(acc[:, :HF] * r + bflat_ref[...], mmean_ref[...],
                 preferred_element_type=jnp.float32)
    x2 = jnp.maximum(x1, 0.0)
    h = jnp.dot(x2, wh_ref[...], preferred_element_type=jnp.float32)
    oh_ref[...] = h.astype(jnp.bfloat16)
    ol_ref[...] = jnp.dot(x2, wl_ref[...], preferred_element_type=jnp.float32)
    or_ref[...] = jnp.dot(x2, wr_ref[...], preferred_element_type=jnp.float32)


def _mid(p0, p1, bmat, bflat, mmean, wh, wl, wr):
    return pl.pallas_call(
        _mid_body,
        grid=(N // MBLK,),
        in_specs=[pl.BlockSpec((MBLK, ROW), lambda i: (i, 0)),
                  pl.BlockSpec((MBLK, ROW), lambda i: (i, 0)),
                  pl.BlockSpec((16, HF), lambda i: (0, 0)),
                  pl.BlockSpec((1, HF), lambda i: (0, 0)),
                  pl.BlockSpec((HF, F), lambda i: (0, 0)),
                  pl.BlockSpec((F, HF), lambda i: (0, 0)),
                  pl.BlockSpec((F, 16), lambda i: (0, 0)),
                  pl.BlockSpec((F, 16), lambda i: (0, 0))],
        out_specs=[pl.BlockSpec((MBLK, HF), lambda i: (i, 0)),
                   pl.BlockSpec((MBLK, 16), lambda i: (i, 0)),
                   pl.BlockSpec((MBLK, 16), lambda i: (i, 0))],
        out_shape=[jax.ShapeDtypeStruct((N, HF), jnp.bfloat16),
                   jax.ShapeDtypeStruct((N, 16), jnp.float32),
                   jax.ShapeDtypeStruct((N, 16), jnp.float32)],
    )(p0, p1, bmat, bflat, mmean, wh, wl, wr)


def _final_body(p0_ref, p1_ref, bmat_ref, bflat_ref, mmean_ref, o_ref):
    acc = p0_ref[...] + p1_ref[...]
    den = jnp.dot(acc[:, HF:ROW], bmat_ref[...],
                  preferred_element_type=jnp.float32)
    r = 1.0 / (den + 1e-9)
    o_ref[...] = jnp.dot(acc[:, :HF] * r + bflat_ref[...], mmean_ref[...],
                         preferred_element_type=jnp.float32)


def _final(p0, p1, bmat, bflat, mmean):
    return pl.pallas_call(
        _final_body,
        grid=(N // MBLK,),
        in_specs=[pl.BlockSpec((MBLK, ROW), lambda i: (i, 0)),
                  pl.BlockSpec((MBLK, ROW), lambda i: (i, 0)),
                  pl.BlockSpec((16, HF), lambda i: (0, 0)),
                  pl.BlockSpec((1, HF), lambda i: (0, 0)),
                  pl.BlockSpec((HF, F), lambda i: (0, 0))],
        out_specs=pl.BlockSpec((MBLK, F), lambda i: (i, 0)),
        out_shape=jax.ShapeDtypeStruct((N, F), jnp.float32),
    )(p0, p1, bmat, bflat, mmean)


def _bcast_lane(v, g):
    return lax.gather(
        v, jnp.full((16, 1), g, jnp.int32),
        lax.GatherDimensionNumbers(offset_dims=(), collapsed_slice_dims=(0,),
                                   start_index_map=(0,)),
        (1,), mode=lax.GatherScatterMode.PROMISE_IN_BOUNDS)


def _edge_pass(src, dst, htab, eltab, ertab):
    mesh = plsc.VectorSubcoreMesh(core_axis_name="c", subcore_axis_name="s")

    def body(src_hbm, dst_hbm, htab_hbm, eltab_hbm, ertab_hbm, out_hbm,
             acc,
             sidx0, sidx1, sidx2, sidx3, didx0, didx1, didx2, didx3,
             hv0, hv1, elv0, elv1, erv0, erv1, msg0, msg1,
             ssi0, ssi1, ssi2, ssi3, sdi0, sdi1, sdi2, sdi3,
             sh0, sh1, sl0, sl1, sre0, sre1, ssc0, ssc1):
        sidx = (sidx0, sidx1, sidx2, sidx3)
        didx = (didx0, didx1, didx2, didx3)
        ssi = (ssi0, ssi1, ssi2, ssi3)
        sdi = (sdi0, sdi1, sdi2, sdi3)
        hv = (hv0, hv1)
        elv = (elv0, elv1)
        erv = (erv0, erv1)
        msg = (msg0, msg1)
        sh = (sh0, sh1)
        sl = (sl0, sl1)
        sre = (sre0, sre1)
        ssc = (ssc0, ssc1)
        c = lax.axis_index("c")
        s = lax.axis_index("s")
        wid = c * 16 + s
        tbase = wid * EPT

        def idx_start(k, b):
            base = pl.multiple_of(tbase + k * CB, 8)
            pltpu.async_copy(src_hbm.at[pl.ds(base, CB)], sidx[b], ssi[b])
            pltpu.async_copy(dst_hbm.at[pl.ds(base, CB)], didx[b], sdi[b])

        def gather_start(b2, bi):
            pltpu.make_async_copy(src_hbm.at[pl.ds(0, CB)], sidx[bi],
                                  ssi[bi]).wait()
            pltpu.make_async_copy(dst_hbm.at[pl.ds(0, CB)], didx[bi],
                                  sdi[bi]).wait()
            pltpu.async_copy(htab_hbm.at[sidx[bi]], hv[b2], sh[b2])
            pltpu.async_copy(eltab_hbm.at[sidx[bi]], elv[b2], sl[b2])
            pltpu.async_copy(ertab_hbm.at[didx[bi]], erv[b2], sre[b2])

        def consume(k, b2, bi):
            hb, eb, rb, mb = hv[b2], elv[b2], erv[b2], msg[b2]
            pltpu.make_async_copy(htab_hbm.at[sidx[bi]], hb, sh[b2]).wait()
            pltpu.make_async_copy(eltab_hbm.at[sidx[bi]], eb, sl[b2]).wait()
            pltpu.make_async_copy(ertab_hbm.at[didx[bi]], rb, sre[b2]).wait()

            @pl.when(k >= 2)
            def _():
                pltpu.make_async_copy(mb, acc.at[didx[bi]], ssc[b2]).wait()

            @plsc.parallel_loop(0, CB, unroll=4)
            def _edge(i):
                el = eb[i, pl.ds(0, 16)]
                er = rb[i, pl.ds(0, 16)]
                e = el + er
                e = jnp.where(e > 0.0, e, 0.2 * e)
                ee = jnp.exp(e)
                mb[i, pl.ds(HF, 16)] = ee
                for b4 in range(4):
                    hp = hb[i, pl.ds(32 * b4, 32)]
                    a0, a1 = plsc.unpack(
                        hp, format=plsc.PackFormat.INTERLEAVED)
                    mb[i, pl.ds(32 * b4, 16)] = a0 * _bcast_lane(ee, 2 * b4)
                    mb[i, pl.ds(32 * b4 + 16, 16)] = (
                        a1 * _bcast_lane(ee, 2 * b4 + 1))

            pltpu.async_copy(mb, acc.at[didx[bi]], ssc[b2], add=True)

        for k in range(4):
            idx_start(k, k)

        def zrow(j, _):
            for kk in range(ROW // 16):
                msg0[j, pl.ds(kk * 16, 16)] = jnp.zeros((16,), jnp.float32)
            return 0
        lax.fori_loop(0, CB, zrow, 0)
        for rblk in range(RPT // CB):
            pltpu.sync_copy(msg0, acc.at[pl.ds(s * RPT + rblk * CB, CB)])
        pltpu.sync_copy(msg0.at[pl.ds(0, RPT % CB)],
                        acc.at[pl.ds(s * RPT + (RPT // CB) * CB, RPT % CB)])

        gather_start(0, 0)
        gather_start(1, 1)

        plsc.subcore_barrier()

        def slot(k, b):
            consume(k, b % 2, b)

            @pl.when(k + 4 < NCHT)
            def _():
                idx_start(k + 4, b)

            @pl.when(k + 2 < NCHT)
            def _():
                gather_start(b % 2, (b + 2) % 4)

        def quad(i, _):
            k0 = 4 * i
            for b in range(4):
                slot(k0 + b, b)
            return 0
        lax.fori_loop(0, NCHT // 4, quad, 0)
        slot(NCHT - 1, (NCHT - 1) % 4)

        pltpu.make_async_copy(msg0, acc.at[didx0], ssc0).wait()
        pltpu.make_async_copy(msg1, acc.at[didx0], ssc1).wait()

        plsc.subcore_barrier()

        pltpu.sync_copy(acc.at[pl.ds(s * RPT, RPT)],
                        out_hbm.at[c, pl.ds(s * RPT, RPT)])

    return pl.kernel(
        body,
        out_type=jax.ShapeDtypeStruct((2, NP, ROW), jnp.float32),
        mesh=mesh,
        compiler_params=pltpu.CompilerParams(use_tc_tiling_on_sc=False,
                                             needs_layout_passes=False),
        scratch_types=(
            [pltpu.VMEM_SHARED((NP, ROW), jnp.float32)]
            + [pltpu.VMEM((CB,), jnp.int32)] * 8
            + [pltpu.VMEM((CB, HF), jnp.bfloat16)] * 2
            + [pltpu.VMEM((CB, 16), jnp.float32)] * 4
            + [pltpu.VMEM((CB, ROW), jnp.float32)] * 2
            + [pltpu.SemaphoreType.DMA] * 16
        ),
    )(src, dst, htab, eltab, ertab)


_PERM = [32 * b + j % 2 * 16 + j // 2 for b in range(4) for j in range(32)]


def _build_weights(W, attn_l, attn_r):
    k = W.shape[0]
    eye_h = jnp.eye(H, dtype=jnp.float32)
    sel_l = (eye_h[:, None, :] * attn_l.T[None, :, :]).reshape(HF, H)
    sel_r = (eye_h[:, None, :] * attn_r.T[None, :, :]).reshape(HF, H)
    z8 = jnp.zeros((k, 8), jnp.float32)
    w_el = jnp.concatenate([W @ sel_l, z8], axis=1)
    w_er = jnp.concatenate([W @ sel_r, z8], axis=1)
    return W[:, jnp.array(_PERM)], w_el, w_er


def kernel(emb, edge_index, W1, attn_l1, attn_r1, b1, W2, attn_l2, attn_r2, b2):
    src = edge_index[0].astype(jnp.int32)
    dst = edge_index[1].astype(jnp.int32)

    wh1, wl1, wr1 = _build_weights(W1, attn_l1, attn_r1)
    wh2, wl2, wr2 = _build_weights(W2, attn_l2, attn_r2)
    bmat = jnp.repeat(jnp.eye(16, dtype=jnp.float32)[:, :H], F, axis=1)
    mmean = jnp.tile(jnp.eye(F, dtype=jnp.float32), (H, 1)) / H
    b1f = b1.reshape(1, HF)
    b2f = b2.reshape(1, HF)

    h1, el1, er1 = _prep(emb, wh1, wl1, wr1)
    part1 = _edge_pass(src, dst, h1, el1, er1)
    h2, el2, er2 = _mid(part1[0, :N], part1[1, :N], bmat, b1f, mmean,
                        wh2, wl2, wr2)
    part2 = _edge_pass(src, dst, h2, el2, er2)
    return _final(part2[0, :N], part2[1, :N], bmat, b2f, mmean)


# speedup vs baseline: 195.4017x; 1.0897x over previous
import jax
import jax.numpy as jnp
from jax import lax
from jax.experimental import pallas as pl
from jax.experimental.pallas import tpu as pltpu
from jax.experimental.pallas import tpu_sc as plsc

N = 10000
E = 320000
H = 8
F = 16
HF = H * F
ROW = HF + 16
TAB = ROW + 16

CB = 80
NTILE = 32
EPT = E // NTILE
NCHT = EPT // CB
NP = 10112
RPT = NP // 16
MBLK = 400


def _matmul_body(x_ref, wh_ref, wl_ref, wr_ref, oh_ref, ol_ref, or_ref):
    x = x_ref[...]
    h = jnp.dot(x, wh_ref[...], preferred_element_type=jnp.float32)
    oh_ref[...] = h.astype(jnp.bfloat16)
    ol_ref[...] = jnp.dot(x, wl_ref[...], preferred_element_type=jnp.float32)
    or_ref[...] = jnp.dot(x, wr_ref[...], preferred_element_type=jnp.float32)


def _prep(x, wh, wl, wr):
    k = x.shape[1]
    return pl.pallas_call(
        _matmul_body,
        grid=(N // MBLK,),
        in_specs=[pl.BlockSpec((MBLK, k), lambda i: (i, 0)),
                  pl.BlockSpec((k, HF), lambda i: (0, 0)),
                  pl.BlockSpec((k, 16), lambda i: (0, 0)),
                  pl.BlockSpec((k, 16), lambda i: (0, 0))],
        out_specs=[pl.BlockSpec((MBLK, HF), lambda i: (i, 0)),
                   pl.BlockSpec((MBLK, 16), lambda i: (i, 0)),
                   pl.BlockSpec((MBLK, 16), lambda i: (i, 0))],
        out_shape=[jax.ShapeDtypeStruct((N, HF), jnp.bfloat16),
                   jax.ShapeDtypeStruct((N, 16), jnp.float32),
                   jax.ShapeDtypeStruct((N, 16), jnp.float32)],
    )(x, wh, wl, wr)


def _mid_body(p0_ref, p1_ref, bmat_ref, bflat_ref, mmean_ref,
              wh_ref, wl_ref, wr_ref, oh_ref, ol_ref, or_ref):
    acc = p0_ref[0] + p1_ref[0]
    den = jnp.dot(acc[:, HF:ROW], bmat_ref[...],
                  preferred_element_type=jnp.float32)
    r = 1.0 / (den + 1e-9)
    x1 = jnp.dot(acc[:, :HF] * r + bflat_ref[...], mmean_ref[...],
                 preferred_element_type=jnp.float32)
    x2 = jnp.maximum(x1, 0.0)
    h = jnp.dot(x2, wh_ref[...], preferred_element_type=jnp.float32)
    oh_ref[...] = h.astype(jnp.bfloat16)
    ol_ref[...] = jnp.dot(x2, wl_ref[...], preferred_element_type=jnp.float32)
    or_ref[...] = jnp.dot(x2, wr_ref[...], preferred_element_type=jnp.float32)


def _mid(part, bmat, bflat, mmean, wh, wl, wr):
    return pl.pallas_call(
        _mid_body,
        grid=(N // MBLK,),
        in_specs=[pl.BlockSpec((1, MBLK, ROW), lambda i: (0, i, 0)),
                  pl.BlockSpec((1, MBLK, ROW), lambda i: (1, i, 0)),
                  pl.BlockSpec((16, HF), lambda i: (0, 0)),
                  pl.BlockSpec((1, HF), lambda i: (0, 0)),
                  pl.BlockSpec((HF, F), lambda i: (0, 0)),
                  pl.BlockSpec((F, HF), lambda i: (0, 0)),
                  pl.BlockSpec((F, 16), lambda i: (0, 0)),
                  pl.BlockSpec((F, 16), lambda i: (0, 0))],
        out_specs=[pl.BlockSpec((MBLK, HF), lambda i: (i, 0)),
                   pl.BlockSpec((MBLK, 16), lambda i: (i, 0)),
                   pl.BlockSpec((MBLK, 16), lambda i: (i, 0))],
        out_shape=[jax.ShapeDtypeStruct((N, HF), jnp.bfloat16),
                   jax.ShapeDtypeStruct((N, 16), jnp.float32),
                   jax.ShapeDtypeStruct((N, 16), jnp.float32)],
    )(part, part, bmat, bflat, mmean, wh, wl, wr)


def _final_body(p0_ref, p1_ref, bmat_ref, bflat_ref, mmean_ref, o_ref):
    acc = p0_ref[0] + p1_ref[0]
    den = jnp.dot(acc[:, HF:ROW], bmat_ref[...],
                  preferred_element_type=jnp.float32)
    r = 1.0 / (den + 1e-9)
    o_ref[...] = jnp.dot(acc[:, :HF] * r + bflat_ref[...], mmean_ref[...],
                         preferred_element_type=jnp.float32)


def _final(part, bmat, bflat, mmean):
    return pl.pallas_call(
        _final_body,
        grid=(N // MBLK,),
        in_specs=[pl.BlockSpec((1, MBLK, ROW), lambda i: (0, i, 0)),
                  pl.BlockSpec((1, MBLK, ROW), lambda i: (1, i, 0)),
                  pl.BlockSpec((16, HF), lambda i: (0, 0)),
                  pl.BlockSpec((1, HF), lambda i: (0, 0)),
                  pl.BlockSpec((HF, F), lambda i: (0, 0))],
        out_specs=pl.BlockSpec((MBLK, F), lambda i: (i, 0)),
        out_shape=jax.ShapeDtypeStruct((N, F), jnp.float32),
    )(part, part, bmat, bflat, mmean)


def _bcast_lane(v, g):
    return lax.gather(
        v, jnp.full((16, 1), g, jnp.int32),
        lax.GatherDimensionNumbers(offset_dims=(), collapsed_slice_dims=(0,),
                                   start_index_map=(0,)),
        (1,), mode=lax.GatherScatterMode.PROMISE_IN_BOUNDS)


def _edge_pass(ei, htab, eltab, ertab):
    mesh = plsc.VectorSubcoreMesh(core_axis_name="c", subcore_axis_name="s")

    def body(ei_hbm, htab_hbm, eltab_hbm, ertab_hbm, out_hbm,
             acc,
             sidx0, sidx1, sidx2, sidx3, didx0, didx1, didx2, didx3,
             hv0, hv1, elv0, elv1, erv0, erv1, msg0, msg1,
             ssi0, ssi1, ssi2, ssi3, sdi0, sdi1, sdi2, sdi3,
             sh0, sh1, sl0, sl1, sre0, sre1, ssc0, ssc1):
        sidx = (sidx0, sidx1, sidx2, sidx3)
        didx = (didx0, didx1, didx2, didx3)
        ssi = (ssi0, ssi1, ssi2, ssi3)
        sdi = (sdi0, sdi1, sdi2, sdi3)
        hv = (hv0, hv1)
        elv = (elv0, elv1)
        erv = (erv0, erv1)
        msg = (msg0, msg1)
        sh = (sh0, sh1)
        sl = (sl0, sl1)
        sre = (sre0, sre1)
        ssc = (ssc0, ssc1)
        c = lax.axis_index("c")
        s = lax.axis_index("s")
        wid = c * 16 + s
        tbase = wid * EPT

        def idx_start(k, b):
            base = pl.multiple_of(tbase + k * CB, 8)
            pltpu.async_copy(ei_hbm.at[0, pl.ds(base, CB)], sidx[b], ssi[b])
            pltpu.async_copy(ei_hbm.at[1, pl.ds(base, CB)], didx[b], sdi[b])

        def gather_start(b2, bi):
            pltpu.make_async_copy(ei_hbm.at[0, pl.ds(0, CB)], sidx[bi],
                                  ssi[bi]).wait()
            pltpu.make_async_copy(ei_hbm.at[1, pl.ds(0, CB)], didx[bi],
                                  sdi[bi]).wait()
            pltpu.async_copy(htab_hbm.at[sidx[bi]], hv[b2], sh[b2])
            pltpu.async_copy(eltab_hbm.at[sidx[bi]], elv[b2], sl[b2])
            pltpu.async_copy(ertab_hbm.at[didx[bi]], erv[b2], sre[b2])

        def consume(k, b2, bi):
            hb, eb, rb, mb = hv[b2], elv[b2], erv[b2], msg[b2]
            pltpu.make_async_copy(htab_hbm.at[sidx[bi]], hb, sh[b2]).wait()
            pltpu.make_async_copy(eltab_hbm.at[sidx[bi]], eb, sl[b2]).wait()
            pltpu.make_async_copy(ertab_hbm.at[didx[bi]], rb, sre[b2]).wait()

            @pl.when(k >= 2)
            def _():
                pltpu.make_async_copy(mb, acc.at[didx[bi]], ssc[b2]).wait()

            @plsc.parallel_loop(0, CB, unroll=4)
            def _edge(i):
                el = eb[i, pl.ds(0, 16)]
                er = rb[i, pl.ds(0, 16)]
                e = el + er
                e = jnp.where(e > 0.0, e, 0.2 * e)
                ee = jnp.exp(e)
                mb[i, pl.ds(HF, 16)] = ee
                for b4 in range(4):
                    hp = hb[i, pl.ds(32 * b4, 32)]
                    a0, a1 = plsc.unpack(
                        hp, format=plsc.PackFormat.INTERLEAVED)
                    mb[i, pl.ds(32 * b4, 16)] = a0 * _bcast_lane(ee, 2 * b4)
                    mb[i, pl.ds(32 * b4 + 16, 16)] = (
                        a1 * _bcast_lane(ee, 2 * b4 + 1))

            pltpu.async_copy(mb, acc.at[didx[bi]], ssc[b2], add=True)

        for k in range(4):
            idx_start(k, k)

        def zrow(j, _):
            for kk in range(ROW // 16):
                msg0[j, pl.ds(kk * 16, 16)] = jnp.zeros((16,), jnp.float32)
            return 0
        lax.fori_loop(0, CB, zrow, 0)
        for rblk in range(RPT // CB):
            pltpu.sync_copy(msg0, acc.at[pl.ds(s * RPT + rblk * CB, CB)])
        pltpu.sync_copy(msg0.at[pl.ds(0, RPT % CB)],
                        acc.at[pl.ds(s * RPT + (RPT // CB) * CB, RPT % CB)])

        gather_start(0, 0)
        gather_start(1, 1)

        plsc.subcore_barrier()

        def slot(k, b):
            consume(k, b % 2, b)

            @pl.when(k + 4 < NCHT)
            def _():
                idx_start(k + 4, b)

            @pl.when(k + 2 < NCHT)
            def _():
                gather_start(b % 2, (b + 2) % 4)

        def quad(i, _):
            k0 = 4 * i
            for b in range(4):
                slot(k0 + b, b)
            return 0
        lax.fori_loop(0, NCHT // 4, quad, 0)
        slot(NCHT - 1, (NCHT - 1) % 4)

        pltpu.make_async_copy(msg0, acc.at[didx0], ssc0).wait()
        pltpu.make_async_copy(msg1, acc.at[didx0], ssc1).wait()

        plsc.subcore_barrier()

        pltpu.sync_copy(acc.at[pl.ds(s * RPT, RPT)],
                        out_hbm.at[c, pl.ds(s * RPT, RPT)])

    return pl.kernel(
        body,
        out_type=jax.ShapeDtypeStruct((2, NP, ROW), jnp.float32),
        mesh=mesh,
        compiler_params=pltpu.CompilerParams(use_tc_tiling_on_sc=False,
                                             needs_layout_passes=False),
        scratch_types=(
            [pltpu.VMEM_SHARED((NP, ROW), jnp.float32)]
            + [pltpu.VMEM((CB,), jnp.int32)] * 8
            + [pltpu.VMEM((CB, HF), jnp.bfloat16)] * 2
            + [pltpu.VMEM((CB, 16), jnp.float32)] * 4
            + [pltpu.VMEM((CB, ROW), jnp.float32)] * 2
            + [pltpu.SemaphoreType.DMA] * 16
        ),
    )(ei, htab, eltab, ertab)


_PERM = [32 * b + j % 2 * 16 + j // 2 for b in range(4) for j in range(32)]


def _build_weights(W, attn_l, attn_r):
    k = W.shape[0]
    eye_h = jnp.eye(H, dtype=jnp.float32)
    sel_l = (eye_h[:, None, :] * attn_l.T[None, :, :]).reshape(HF, H)
    sel_r = (eye_h[:, None, :] * attn_r.T[None, :, :]).reshape(HF, H)
    z8 = jnp.zeros((k, 8), jnp.float32)
    w_el = jnp.concatenate([W @ sel_l, z8], axis=1)
    w_er = jnp.concatenate([W @ sel_r, z8], axis=1)
    return W[:, jnp.array(_PERM)], w_el, w_er


def kernel(emb, edge_index, W1, attn_l1, attn_r1, b1, W2, attn_l2, attn_r2, b2):
    ei = edge_index.astype(jnp.int32)

    wh1, wl1, wr1 = _build_weights(W1, attn_l1, attn_r1)
    wh2, wl2, wr2 = _build_weights(W2, attn_l2, attn_r2)
    bmat = jnp.repeat(jnp.eye(16, dtype=jnp.float32)[:, :H], F, axis=1)
    mmean = jnp.tile(jnp.eye(F, dtype=jnp.float32), (H, 1)) / H
    b1f = b1.reshape(1, HF)
    b2f = b2.reshape(1, HF)

    h1, el1, er1 = _prep(emb, wh1, wl1, wr1)
    part1 = _edge_pass(ei, h1, el1, er1)
    h2, el2, er2 = _mid(part1, bmat, b1f, mmean, wh2, wl2, wr2)
    part2 = _edge_pass(ei, h2, el2, er2)
    return _final(part2, bmat, b2f, mmean)
